# baseline (speedup 1.0000x reference)
import functools

import jax
import jax.numpy as jnp
from jax import lax
from jax.experimental import pallas as pl
from jax.experimental.pallas import tpu as pltpu
from jax.experimental.pallas import tpu_sc as plsc

_N, _D, _H, _E, _TOPK = 32768, 768, 128, 64, 8
_BN = 1024
_C = 4
_NCHUNK = _N // _C

_NC, _NS, _L = 2, 16, 16
_NW = _NC * _NS
_RW = _NCHUNK // _NW


def _logits_body(x_ref, w1_ref, b1_ref, w2_ref, b2_ref,
                 logits_ref, logits_t_ref):
    h = jnp.dot(x_ref[...], w1_ref[...], preferred_element_type=jnp.float32)
    h = h + b1_ref[...]
    h = h * jax.nn.sigmoid(h)
    w2 = w2_ref[...]
    b2 = b2_ref[...]
    logits = jnp.dot(h, w2, preferred_element_type=jnp.float32)
    logits_ref[...] = logits + b2
    lt = lax.dot_general(w2, h, (((0,), (1,)), ((), ())),
                         preferred_element_type=jnp.float32)
    logits_t_ref[...] = lt + b2.reshape(_E, 1)


def _tc_logits(x, W1, b1r, W2, b2r):
    return pl.pallas_call(
        _logits_body,
        grid=(_NCHUNK // _BN,),
        in_specs=[
            pl.BlockSpec((_BN, _D), lambda i: (i, 0)),
            pl.BlockSpec((_D, _H), lambda i: (0, 0)),
            pl.BlockSpec((1, _H), lambda i: (0, 0)),
            pl.BlockSpec((_H, _E), lambda i: (0, 0)),
            pl.BlockSpec((1, _E), lambda i: (0, 0)),
        ],
        out_specs=[
            pl.BlockSpec((_BN, _E), lambda i: (i, 0)),
            pl.BlockSpec((_E, _BN), lambda i: (0, i)),
        ],
        out_shape=[
            jax.ShapeDtypeStruct((_NCHUNK, _E), jnp.float32),
            jax.ShapeDtypeStruct((_E, _NCHUNK), jnp.float32),
        ],
    )(x, W1, b1r, W2, b2r)


def _sc_topk_body(lt_hbm, kl_hbm, kp_hbm, ki_hbm, lg_v, kl_v, kp_v, ki_v):
    wid = lax.axis_index("s") * _NC + lax.axis_index("c")
    base = wid * _RW
    pltpu.sync_copy(lt_hbm.at[:, pl.ds(base, _RW)], lg_v)

    neg_inf = jnp.full((_L,), -jnp.inf, jnp.float32)
    zero_i = jnp.zeros((_L,), jnp.int32)

    def group(g, carry):
        t0 = g * _L
        vs = [neg_inf] * _TOPK
        ix = [zero_i] * _TOPK
        for e in range(_E):
            nv = lg_v[e, pl.ds(t0, _L)]
            ne = jnp.full((_L,), e, jnp.int32)
            cs = [nv > vs[j] for j in range(_TOPK)]
            nvs, nis = [], []
            for j in range(_TOPK):
                if j == 0:
                    nvs.append(jnp.where(cs[0], nv, vs[0]))
                    nis.append(jnp.where(cs[0], ne, ix[0]))
                else:
                    innerv = jnp.where(cs[j - 1], vs[j - 1], nv)
                    inneri = jnp.where(cs[j - 1], ix[j - 1], ne)
                    nvs.append(jnp.where(cs[j], innerv, vs[j]))
                    nis.append(jnp.where(cs[j], inneri, ix[j]))
            vs, ix = nvs, nis
        ps = [jnp.exp(v - vs[0]) for v in vs]
        tot = ps[0]
        for j in range(1, _TOPK):
            tot = tot + ps[j]
        inv = 1.0 / tot
        for j in range(_TOPK):
            kl_v[j, pl.ds(t0, _L)] = vs[j]
            kp_v[j, pl.ds(t0, _L)] = ps[j] * inv
            ki_v[j, pl.ds(t0, _L)] = ix[j]
        return carry

    lax.fori_loop(0, _RW // _L, group, 0)

    pltpu.sync_copy(kl_v, kl_hbm.at[:, pl.ds(base, _RW)])
    pltpu.sync_copy(kp_v, kp_hbm.at[:, pl.ds(base, _RW)])
    pltpu.sync_copy(ki_v, ki_hbm.at[:, pl.ds(base, _RW)])


_sc_topk = functools.partial(
    pl.kernel,
    _sc_topk_body,
    out_type=[
        jax.ShapeDtypeStruct((_TOPK, _NCHUNK), jnp.float32),
        jax.ShapeDtypeStruct((_TOPK, _NCHUNK), jnp.float32),
        jax.ShapeDtypeStruct((_TOPK, _NCHUNK), jnp.int32),
    ],
    mesh=plsc.VectorSubcoreMesh(
        core_axis_name="c", subcore_axis_name="s",
        num_cores=_NC, num_subcores=_NS,
    ),
    scratch_types=[
        pltpu.VMEM((_E, _RW), jnp.float32),
        pltpu.VMEM((_TOPK, _RW), jnp.float32),
        pltpu.VMEM((_TOPK, _RW), jnp.float32),
        pltpu.VMEM((_TOPK, _RW), jnp.int32),
    ],
)


def kernel(input, W1, b1, W2, b2):
    b1r = b1.reshape(1, _H)
    b2r = b2.reshape(1, _E)
    logits_parts = []
    kl_parts, kp_parts, ki_parts = [], [], []
    for c in range(_C):
        xc = lax.slice_in_dim(input, c * _NCHUNK, (c + 1) * _NCHUNK, axis=0)
        logits_c, logits_t_c = _tc_logits(xc, W1, b1r, W2, b2r)
        kl_t, kp_t, ki_t = _sc_topk()(logits_t_c)
        logits_parts.append(logits_c)
        kl_parts.append(kl_t)
        kp_parts.append(kp_t)
        ki_parts.append(ki_t)
    logits = jnp.concatenate(logits_parts, axis=0)
    kl = jnp.concatenate(kl_parts, axis=1).T
    kp = jnp.concatenate(kp_parts, axis=1).T
    ki = jnp.concatenate(ki_parts, axis=1).T
    return (logits, kl, kp, ki)


# speedup vs baseline: 1.6932x; 1.6932x over previous
import functools

import jax
import jax.numpy as jnp
from jax import lax
from jax.experimental import pallas as pl
from jax.experimental.pallas import tpu as pltpu
from jax.experimental.pallas import tpu_sc as plsc

_N, _D, _H, _E, _TOPK = 32768, 768, 128, 64, 8
_BN = 1024
_C = 4
_NCHUNK = _N // _C
_GC = _NCHUNK // _BN

_NC, _NS, _L = 2, 16, 16
_NW = _NC * _NS
_RW = _NCHUNK // _NW


def _logits_body_first(x_ref, w1_ref, b1_ref, w2_ref, b2_ref,
                       logits_ref, logits_t_ref):
    h = jnp.dot(x_ref[...], w1_ref[...], preferred_element_type=jnp.float32)
    h = h + b1_ref[...]
    h = h * jax.nn.sigmoid(h)
    w2 = w2_ref[...]
    b2 = b2_ref[...]
    logits = jnp.dot(h, w2, preferred_element_type=jnp.float32)
    logits_ref[...] = logits + b2
    lt = lax.dot_general(w2, h, (((0,), (1,)), ((), ())),
                         preferred_element_type=jnp.float32)
    logits_t_ref[...] = lt + b2.reshape(_E, 1)


def _logits_body_rest(prev_ref, x_ref, w1_ref, b1_ref, w2_ref, b2_ref,
                      logits_ref, logits_t_ref):
    del prev_ref
    _logits_body_first(x_ref, w1_ref, b1_ref, w2_ref, b2_ref,
                       logits_ref, logits_t_ref)


def _tc_logits_chunk(c, x, W1, b1r, W2, b2r, logits_prev):
    x_spec = pl.BlockSpec((_BN, _D), lambda i, c=c: (c * _GC + i, 0))
    w_specs = [
        pl.BlockSpec((_D, _H), lambda i: (0, 0)),
        pl.BlockSpec((1, _H), lambda i: (0, 0)),
        pl.BlockSpec((_H, _E), lambda i: (0, 0)),
        pl.BlockSpec((1, _E), lambda i: (0, 0)),
    ]
    out_specs = [
        pl.BlockSpec((_BN, _E), lambda i, c=c: (c * _GC + i, 0)),
        pl.BlockSpec((_E, _BN), lambda i: (0, i)),
    ]
    out_shape = [
        jax.ShapeDtypeStruct((_N, _E), jnp.float32),
        jax.ShapeDtypeStruct((_E, _NCHUNK), jnp.float32),
    ]
    if c == 0:
        return pl.pallas_call(
            _logits_body_first,
            grid=(_GC,),
            in_specs=[x_spec] + w_specs,
            out_specs=out_specs,
            out_shape=out_shape,
        )(x, W1, b1r, W2, b2r)
    prev_spec = pl.BlockSpec((_BN, _E), lambda i, c=c: (c * _GC + i, 0))
    return pl.pallas_call(
        _logits_body_rest,
        grid=(_GC,),
        in_specs=[prev_spec, x_spec] + w_specs,
        out_specs=out_specs,
        out_shape=out_shape,
        input_output_aliases={0: 0},
    )(logits_prev, x, W1, b1r, W2, b2r)


def _sc_topk_body(lt_hbm, kl_hbm, kp_hbm, ki_hbm, lg_v, kl_v, kp_v, ki_v):
    wid = lax.axis_index("s") * _NC + lax.axis_index("c")
    base = wid * _RW
    pltpu.sync_copy(lt_hbm.at[:, pl.ds(base, _RW)], lg_v)

    neg_inf = jnp.full((_L,), -jnp.inf, jnp.float32)
    zero_i = jnp.zeros((_L,), jnp.int32)

    def group(g, carry):
        t0 = g * _L
        vs = [neg_inf] * _TOPK
        ix = [zero_i] * _TOPK
        for e in range(_E):
            nv = lg_v[e, pl.ds(t0, _L)]
            ne = jnp.full((_L,), e, jnp.int32)
            cs = [nv > vs[j] for j in range(_TOPK)]
            nvs, nis = [], []
            for j in range(_TOPK):
                if j == 0:
                    nvs.append(jnp.where(cs[0], nv, vs[0]))
                    nis.append(jnp.where(cs[0], ne, ix[0]))
                else:
                    innerv = jnp.where(cs[j - 1], vs[j - 1], nv)
                    inneri = jnp.where(cs[j - 1], ix[j - 1], ne)
                    nvs.append(jnp.where(cs[j], innerv, vs[j]))
                    nis.append(jnp.where(cs[j], inneri, ix[j]))
            vs, ix = nvs, nis
        ps = [jnp.exp(v - vs[0]) for v in vs]
        tot = ps[0]
        for j in range(1, _TOPK):
            tot = tot + ps[j]
        inv = 1.0 / tot
        for j in range(_TOPK):
            kl_v[j, pl.ds(t0, _L)] = vs[j]
            kp_v[j, pl.ds(t0, _L)] = ps[j] * inv
            ki_v[j, pl.ds(t0, _L)] = ix[j]
        return carry

    lax.fori_loop(0, _RW // _L, group, 0)

    pltpu.sync_copy(kl_v, kl_hbm.at[:, pl.ds(base, _RW)])
    pltpu.sync_copy(kp_v, kp_hbm.at[:, pl.ds(base, _RW)])
    pltpu.sync_copy(ki_v, ki_hbm.at[:, pl.ds(base, _RW)])


_sc_topk = functools.partial(
    pl.kernel,
    _sc_topk_body,
    out_type=[
        jax.ShapeDtypeStruct((_TOPK, _NCHUNK), jnp.float32),
        jax.ShapeDtypeStruct((_TOPK, _NCHUNK), jnp.float32),
        jax.ShapeDtypeStruct((_TOPK, _NCHUNK), jnp.int32),
    ],
    mesh=plsc.VectorSubcoreMesh(
        core_axis_name="c", subcore_axis_name="s",
        num_cores=_NC, num_subcores=_NS,
    ),
    scratch_types=[
        pltpu.VMEM((_E, _RW), jnp.float32),
        pltpu.VMEM((_TOPK, _RW), jnp.float32),
        pltpu.VMEM((_TOPK, _RW), jnp.float32),
        pltpu.VMEM((_TOPK, _RW), jnp.int32),
    ],
)


def kernel(input, W1, b1, W2, b2):
    b1r = b1.reshape(1, _H)
    b2r = b2.reshape(1, _E)
    logits = None
    kl_parts, kp_parts, ki_parts = [], [], []
    for c in range(_C):
        logits, logits_t_c = _tc_logits_chunk(c, input, W1, b1r, W2, b2r, logits)
        kl_t, kp_t, ki_t = _sc_topk()(logits_t_c)
        kl_parts.append(kl_t)
        kp_parts.append(kp_t)
        ki_parts.append(ki_t)
    kl = jnp.concatenate(kl_parts, axis=1).T
    kp = jnp.concatenate(kp_parts, axis=1).T
    ki = jnp.concatenate(ki_parts, axis=1).T
    return (logits, kl, kp, ki)
